# baseline (speedup 1.0000x reference)
import functools

import jax
import jax.numpy as jnp
from jax import lax
from jax.experimental import pallas as pl
from jax.experimental.pallas import tpu as pltpu


BM = 128
FF = 1024


def _ffn_body(be_ref, valid_ref, xg_ref, w1_ref, w3_ref, w2_ref, out_ref,
              acc_ref):
    f = pl.program_id(0)
    m = pl.program_id(1)
    nf = pl.num_programs(0)

    @pl.when(f == 0)
    def _init():
        acc_ref[pl.ds(m * BM, BM), :] = jnp.zeros((BM, acc_ref.shape[1]),
                                                  jnp.float32)

    @pl.when(valid_ref[m] > 0)
    def _compute():
        xb = xg_ref[...].astype(jnp.bfloat16)
        w1b = w1_ref[0].astype(jnp.bfloat16)
        w3b = w3_ref[0].astype(jnp.bfloat16)
        w2b = w2_ref[0].astype(jnp.bfloat16)
        dn = (((1,), (1,)), ((), ()))
        x1 = lax.dot_general(xb, w1b, dn, preferred_element_type=jnp.float32)
        x3 = lax.dot_general(xb, w3b, dn, preferred_element_type=jnp.float32)
        h = (x1 * jax.nn.sigmoid(x1) * x3).astype(jnp.bfloat16)
        contrib = lax.dot_general(h, w2b, dn,
                                  preferred_element_type=jnp.float32)
        acc_ref[pl.ds(m * BM, BM), :] += contrib

    @pl.when(f == nf - 1)
    def _write():
        out_ref[...] = acc_ref[pl.ds(m * BM, BM), :]


def _grouped_ffn(xg, w1, w3, w2, be, valid, cap_rows):
    num_e, inter, dim = w1.shape
    m_blocks = cap_rows // BM
    f_blocks = inter // FF
    grid_spec = pltpu.PrefetchScalarGridSpec(
        num_scalar_prefetch=2,
        grid=(f_blocks, m_blocks),
        in_specs=[
            pl.BlockSpec((BM, dim), lambda f, m, be, va: (m, 0)),
            pl.BlockSpec((1, FF, dim), lambda f, m, be, va: (be[m], f, 0)),
            pl.BlockSpec((1, FF, dim), lambda f, m, be, va: (be[m], f, 0)),
            pl.BlockSpec((1, dim, FF), lambda f, m, be, va: (be[m], 0, f)),
        ],
        out_specs=pl.BlockSpec((BM, dim), lambda f, m, be, va: (m, 0)),
        scratch_shapes=[pltpu.VMEM((cap_rows, dim), jnp.float32)],
    )
    return pl.pallas_call(
        _ffn_body,
        grid_spec=grid_spec,
        out_shape=jax.ShapeDtypeStruct((cap_rows, dim), jnp.float32),
        compiler_params=pltpu.CompilerParams(
            dimension_semantics=("arbitrary", "arbitrary")),
    )(be, valid, xg, w1, w3, w2)


def kernel(x, expert_indices, w1, w2, w3):
    seq_len, dim = x.shape
    top_k = expert_indices.shape[1]
    num_e = w1.shape[0]
    p = seq_len * top_k
    cap_rows = p + num_e * BM
    m_blocks = cap_rows // BM

    e_flat = expert_indices.reshape(-1).astype(jnp.int32)
    oh = (e_flat[:, None] == jnp.arange(num_e, dtype=jnp.int32)[None, :])
    oh = oh.astype(jnp.int32)
    counts = oh.sum(0)
    nb = (counts + BM - 1) // BM
    starts_blk = jnp.concatenate(
        [jnp.zeros((1,), jnp.int32), jnp.cumsum(nb)[:-1].astype(jnp.int32)])
    rank = (jnp.cumsum(oh, axis=0) * oh).sum(1) - 1
    pos = starts_blk[e_flat] * BM + rank
    total_blk = nb.sum()
    bids = jnp.arange(m_blocks, dtype=jnp.int32)
    be = jnp.searchsorted(starts_blk, bids, side="right").astype(jnp.int32) - 1
    e_last = (jnp.searchsorted(starts_blk, total_blk - 1, side="right")
              .astype(jnp.int32) - 1)
    be = jnp.where(bids < total_blk, be, e_last).astype(jnp.int32)
    valid = (bids < total_blk).astype(jnp.int32)
    tok = jnp.arange(p, dtype=jnp.int32) // top_k
    tok_padded = jnp.zeros((cap_rows,), jnp.int32).at[pos].set(tok)

    xg = x[tok_padded]

    y = _grouped_ffn(xg, w1, w3, w2, be, valid, cap_rows)

    out = y[pos]
    return out.reshape(seq_len, top_k, dim)
